# Initial kernel scaffold; baseline (speedup 1.0000x reference)
import jax
import jax.numpy as jnp
from jax.experimental import pallas as pl


def kernel(vals, time, masks, lengths, l_t, W, b):
    raise NotImplementedError("write your pallas kernel here")


# speedup vs baseline: 216.4242x; 216.4242x over previous
import functools

import jax
import jax.numpy as jnp
from jax import lax
from jax.experimental import pallas as pl
from jax.experimental.pallas import tpu as pltpu
from jax.experimental.pallas import tpu_sc as plsc

_B, _T, _V = 8, 2048, 32
_NG = 16
_NQ = 2 * _NG
_L = 16
_NW = 32
_PPW = _B * _V // _NW
_VPW = _V // (_NW // _B)


def _sc_glimpse(vals_t, time, masks_t, l_b):
    mesh = plsc.VectorSubcoreMesh(core_axis_name="c", subcore_axis_name="s")

    @functools.partial(
        pl.kernel,
        out_type=jax.ShapeDtypeStruct((_B, _V * _NQ), jnp.float32),
        mesh=mesh,
        compiler_params=pltpu.CompilerParams(needs_layout_passes=False),
        scratch_types=[
            pltpu.VMEM((_T,), jnp.float32),
            pltpu.VMEM((_T,), jnp.float32),
            pltpu.VMEM((_T,), jnp.int32),
            pltpu.VMEM((_T + _L,), jnp.float32),
            pltpu.VMEM((_T + _L,), jnp.float32),
            pltpu.VMEM((_L,), jnp.float32),
            pltpu.VMEM((_VPW * _NQ,), jnp.float32),
        ],
    )
    def body(vals_hbm, time_hbm, masks_hbm, l_hbm, out_hbm,
             t_v, x_v, m_v, obs_t, obs_v, l_v, o_v):
        wid = lax.axis_index("s") * 2 + lax.axis_index("c")
        b = wid // (_NW // _B)
        v0 = (wid % (_NW // _B)) * _VPW
        pltpu.sync_copy(time_hbm.at[b], t_v)
        pltpu.sync_copy(l_hbm.at[b], l_v)
        lvec = l_v[...]
        iotaf = lax.iota(jnp.int32, _L).astype(jnp.float32)
        lin0 = iotaf * jnp.float32(0.1 / 15.0) + jnp.float32(-0.05)
        lin1 = iotaf * jnp.float32(0.5 / 15.0) + jnp.float32(-0.25)

        def pair_body(j, carry):
            v = v0 + j
            pltpu.sync_copy(vals_hbm.at[b, v], x_v)
            pltpu.sync_copy(masks_hbm.at[b, v], m_v)

            def comp_body(i, cnt):
                off = i * _L
                mi = m_v[pl.ds(off, _L)]
                mm = mi != 0
                dest = jnp.full((_L,), cnt, jnp.int32) + jnp.cumsum(mi) - 1
                plsc.store_scatter(obs_t, [dest], t_v[pl.ds(off, _L)], mask=mm)
                plsc.store_scatter(obs_v, [dest], x_v[pl.ds(off, _L)], mask=mm)
                return cnt + jnp.sum(mi)

            n_obs = lax.fori_loop(0, _T // _L, comp_body, jnp.int32(0))
            nvec = jnp.full((_L,), n_obs, jnp.int32)
            last_idx = jnp.maximum(nvec - 1, 0)
            last_t = plsc.load_gather(obs_t, [last_idx])
            last_v = plsc.load_gather(obs_v, [last_idx])
            idx_hi = jnp.maximum(nvec - 2, 0)

            def interp(lin):
                r = (lin + lvec) * last_t
                cnt = jnp.zeros((_L,), jnp.int32)
                step = _T
                while step >= 1:
                    cand = cnt + step
                    gidx = jnp.minimum(cand - 1, last_idx)
                    tv = plsc.load_gather(obs_t, [gidx])
                    ok = (cand <= nvec) & (tv <= r)
                    cnt = jnp.where(ok, cand, cnt)
                    step //= 2
                idx0 = jnp.clip(cnt - 1, 0, idx_hi)
                idx1 = idx0 + 1
                x0 = plsc.load_gather(obs_t, [idx0])
                x1 = plsc.load_gather(obs_t, [idx1])
                y0 = plsc.load_gather(obs_v, [idx0])
                y1 = plsc.load_gather(obs_v, [idx1])
                y = y0 + (r - x0) / (x1 - x0) * (y1 - y0)
                y = jnp.where(r < x0, y0, y)
                y = jnp.where(r > last_t, last_v, y)
                y = jnp.where(nvec == 1, last_v, y)
                y = jnp.where(nvec == 0, jnp.zeros((_L,), jnp.float32), y)
                return y

            o_v[pl.ds(j * _NQ, _L)] = interp(lin0)
            o_v[pl.ds(j * _NQ + _NG, _L)] = interp(lin1)
            return carry

        lax.fori_loop(0, _VPW, pair_body, jnp.int32(0))
        pltpu.sync_copy(o_v, out_hbm.at[b, pl.ds(v0 * _NQ, _VPW * _NQ)])

    return body(vals_t, time, masks_t, l_b)


def _tc_fc(g, l_t, wgt, wl, brow, nhid):
    def fc_body(g_ref, l_ref, w_ref, wl_ref, b_ref, o_ref):
        o_ref[...] = (
            jnp.dot(g_ref[...], w_ref[...], preferred_element_type=jnp.float32)
            + l_ref[...] * wl_ref[...]
            + b_ref[...]
        )

    return pl.pallas_call(
        fc_body,
        out_shape=jax.ShapeDtypeStruct((_B, nhid), jnp.float32),
    )(g, l_t, wgt, wl, brow)


def kernel(vals, time, masks, lengths, l_t, W, b):
    del lengths
    nhid = W.shape[0]
    vals_t = jnp.transpose(vals, (0, 2, 1))
    masks_t = jnp.transpose(masks, (0, 2, 1)).astype(jnp.int32)
    l_b = jnp.broadcast_to(l_t, (_B, _L))
    g = _sc_glimpse(vals_t, time, masks_t, l_b)
    wgt = W[:, :-1].T
    wl = W[:, -1].reshape(1, nhid)
    grep = _tc_fc(g, l_t, wgt, wl, b.reshape(1, nhid), nhid)
    return grep, g[:, g.shape[1] // 2]
